# Initial kernel scaffold; baseline (speedup 1.0000x reference)
import jax
import jax.numpy as jnp
from jax.experimental import pallas as pl


def kernel(xys, xylens, rgbs, embedding):
    raise NotImplementedError("write your pallas kernel here")


# speedup vs baseline: 6.1146x; 6.1146x over previous
import functools

import jax
import jax.numpy as jnp
from jax import lax
from jax.experimental import pallas as pl
from jax.experimental.pallas import tpu as pltpu
from jax.experimental.pallas import tpu_sc as plsc

BATCH = 16384
FIELD = 32
EMB_DIM = 64
TOTAL = BATCH * 3 * FIELD
CHUNK = 128
NROWS = TOTAL // CHUNK
NW = 32
ROWS_PER_W = NROWS // NW


def _make_kernel():
    mesh = plsc.VectorSubcoreMesh(
        core_axis_name="c", subcore_axis_name="s", num_cores=2, num_subcores=16
    )

    @functools.partial(
        pl.kernel,
        out_type=jax.ShapeDtypeStruct((TOTAL, EMB_DIM), jnp.float32),
        mesh=mesh,
        scratch_types=[
            pltpu.VMEM((ROWS_PER_W, CHUNK), jnp.int32),
            pltpu.VMEM((CHUNK, EMB_DIM), jnp.float32),
            pltpu.SemaphoreType.DMA,
        ],
        compiler_params=pltpu.CompilerParams(use_tc_tiling_on_sc=False),
    )
    def gather_kernel(idx_hbm, table_hbm, out_hbm, idx_v, rows_v, sem):
        wid = lax.axis_index("s") * 2 + lax.axis_index("c")
        base = wid * ROWS_PER_W
        pltpu.sync_copy(idx_hbm.at[pl.ds(base, ROWS_PER_W), :], idx_v)

        def chunk_body(c, carry):
            pltpu.async_copy(table_hbm.at[idx_v.at[c]], rows_v, sem).wait()
            pltpu.sync_copy(
                rows_v, out_hbm.at[pl.ds((base + c) * CHUNK, CHUNK), :]
            )
            return carry

        lax.fori_loop(0, ROWS_PER_W, chunk_body, 0, unroll=False)

    return gather_kernel


_gather = _make_kernel()


def kernel(xys, xylens, rgbs, embedding):
    if xys.ndim == 3:
        xys = xys.reshape(xys.shape[0], -1)
    if xylens.ndim == 3:
        xylens = xylens.reshape(xylens.shape[0], -1)
    if rgbs.ndim == 3:
        rgbs = rgbs.reshape(rgbs.shape[0], -1)
    everything = jnp.concatenate((xys, xylens, rgbs), axis=-1)
    idx = everything.reshape(NROWS, CHUNK)
    out = _gather(idx, embedding)
    return out.reshape(xys.shape[0], -1)


# speedup vs baseline: 6.6464x; 1.0870x over previous
import functools

import jax
import jax.numpy as jnp
from jax import lax
from jax.experimental import pallas as pl
from jax.experimental.pallas import tpu as pltpu
from jax.experimental.pallas import tpu_sc as plsc

BATCH = 16384
FIELD = 32
EMB_DIM = 64
TOTAL = BATCH * 3 * FIELD
CHUNK = 128
NROWS = TOTAL // CHUNK
NW = 32
ROWS_PER_W = NROWS // NW
GPC = 4
SC_ROWS = CHUNK * GPC
NSC = ROWS_PER_W // GPC
NSTEP = NSC // 2


def _make_kernel():
    mesh = plsc.VectorSubcoreMesh(
        core_axis_name="c", subcore_axis_name="s", num_cores=2, num_subcores=16
    )

    @functools.partial(
        pl.kernel,
        out_type=jax.ShapeDtypeStruct((TOTAL, EMB_DIM), jnp.float32),
        mesh=mesh,
        scratch_types=[
            pltpu.VMEM((ROWS_PER_W, CHUNK), jnp.int32),
            pltpu.VMEM((2, SC_ROWS, EMB_DIM), jnp.float32),
            pltpu.SemaphoreType.DMA,
            pltpu.SemaphoreType.DMA,
            pltpu.SemaphoreType.DMA,
            pltpu.SemaphoreType.DMA,
        ],
        compiler_params=pltpu.CompilerParams(use_tc_tiling_on_sc=False),
    )
    def gather_kernel(idx_hbm, table_hbm, out_hbm, idx_v, rows_v, g0, g1, s0, s1):
        wid = lax.axis_index("s") * 2 + lax.axis_index("c")
        base = wid * ROWS_PER_W
        out_base = wid * NSC
        pltpu.sync_copy(idx_hbm.at[pl.ds(base, ROWS_PER_W), :], idx_v)
        g_sems = (g0, g1)
        s_sems = (s0, s1)

        def half(step, b):
            c = step * 2 + b
            buf = rows_v.at[b]
            out_slc = out_hbm.at[pl.ds((out_base + c) * SC_ROWS, SC_ROWS), :]

            @pl.when(step > 0)
            def _():
                pltpu.make_async_copy(buf, out_slc, s_sems[b]).wait()

            handles = [
                pltpu.async_copy(
                    table_hbm.at[idx_v.at[c * GPC + k]],
                    rows_v.at[b, pl.ds(k * CHUNK, CHUNK), :],
                    g_sems[b],
                )
                for k in range(GPC)
            ]
            for h in handles:
                h.wait()
            pltpu.async_copy(buf, out_slc, s_sems[b])

        def step_body(step, carry):
            half(step, 0)
            half(step, 1)
            return carry

        lax.fori_loop(0, NSTEP, step_body, 0, unroll=False)

        for b in range(2):
            c = NSC - 2 + b
            pltpu.make_async_copy(
                rows_v.at[b],
                out_hbm.at[pl.ds((out_base + c) * SC_ROWS, SC_ROWS), :],
                s_sems[b],
            ).wait()

    return gather_kernel


_gather = _make_kernel()


def kernel(xys, xylens, rgbs, embedding):
    if xys.ndim == 3:
        xys = xys.reshape(xys.shape[0], -1)
    if xylens.ndim == 3:
        xylens = xylens.reshape(xylens.shape[0], -1)
    if rgbs.ndim == 3:
        rgbs = rgbs.reshape(rgbs.shape[0], -1)
    everything = jnp.concatenate((xys, xylens, rgbs), axis=-1)
    idx = everything.reshape(NROWS, CHUNK)
    out = _gather(idx, embedding)
    return out.reshape(xys.shape[0], -1)


# speedup vs baseline: 11.5348x; 1.7355x over previous
import functools

import jax
import jax.numpy as jnp
from jax import lax
from jax.experimental import pallas as pl
from jax.experimental.pallas import tpu as pltpu
from jax.experimental.pallas import tpu_sc as plsc

BATCH = 16384
FIELD = 32
EMB_DIM = 64
TOTAL = BATCH * 3 * FIELD
CHUNK = 128
NROWS = TOTAL // CHUNK
NW = 32
ROWS_PER_W = NROWS // NW
GPC = 4
SC_ROWS = CHUNK * GPC
NSC = ROWS_PER_W // GPC
NSTEP = NSC // 2


def _make_kernel():
    mesh = plsc.VectorSubcoreMesh(
        core_axis_name="c", subcore_axis_name="s", num_cores=2, num_subcores=16
    )

    @functools.partial(
        pl.kernel,
        out_type=jax.ShapeDtypeStruct((TOTAL, EMB_DIM), jnp.float32),
        mesh=mesh,
        scratch_types=[
            pltpu.VMEM((ROWS_PER_W, CHUNK), jnp.int32),
            pltpu.VMEM((2, SC_ROWS, EMB_DIM), jnp.float32),
            pltpu.VMEM_SHARED((1000, EMB_DIM), jnp.float32),
            pltpu.SemaphoreType.DMA,
            pltpu.SemaphoreType.DMA,
            pltpu.SemaphoreType.DMA,
            pltpu.SemaphoreType.DMA,
        ],
        compiler_params=pltpu.CompilerParams(use_tc_tiling_on_sc=False),
    )
    def gather_kernel(
        idx_hbm, table_hbm, out_hbm, idx_v, rows_v, tab_sh, g0, g1, s0, s1
    ):
        sid = lax.axis_index("s")
        wid = sid * 2 + lax.axis_index("c")
        base = wid * ROWS_PER_W
        out_base = wid * NSC

        @pl.when(sid == 0)
        def _():
            pltpu.sync_copy(table_hbm, tab_sh)

        pltpu.sync_copy(idx_hbm.at[pl.ds(base, ROWS_PER_W), :], idx_v)
        plsc.subcore_barrier()
        g_sems = (g0, g1)
        s_sems = (s0, s1)

        def half(step, b):
            c = step * 2 + b
            buf = rows_v.at[b]
            out_slc = out_hbm.at[pl.ds((out_base + c) * SC_ROWS, SC_ROWS), :]

            @pl.when(step > 0)
            def _():
                pltpu.make_async_copy(buf, out_slc, s_sems[b]).wait()

            handles = [
                pltpu.async_copy(
                    tab_sh.at[idx_v.at[c * GPC + k]],
                    rows_v.at[b, pl.ds(k * CHUNK, CHUNK), :],
                    g_sems[b],
                )
                for k in range(GPC)
            ]
            for h in handles:
                h.wait()
            pltpu.async_copy(buf, out_slc, s_sems[b])

        def step_body(step, carry):
            half(step, 0)
            half(step, 1)
            return carry

        lax.fori_loop(0, NSTEP, step_body, 0, unroll=False)

        for b in range(2):
            c = NSC - 2 + b
            pltpu.make_async_copy(
                rows_v.at[b],
                out_hbm.at[pl.ds((out_base + c) * SC_ROWS, SC_ROWS), :],
                s_sems[b],
            ).wait()

    return gather_kernel


_gather = _make_kernel()


def kernel(xys, xylens, rgbs, embedding):
    if xys.ndim == 3:
        xys = xys.reshape(xys.shape[0], -1)
    if xylens.ndim == 3:
        xylens = xylens.reshape(xylens.shape[0], -1)
    if rgbs.ndim == 3:
        rgbs = rgbs.reshape(rgbs.shape[0], -1)
    everything = jnp.concatenate((xys, xylens, rgbs), axis=-1)
    idx = everything.reshape(NROWS, CHUNK)
    out = _gather(idx, embedding)
    return out.reshape(xys.shape[0], -1)
